# baseline (speedup 1.0000x reference)
import functools

import jax
import jax.numpy as jnp
from jax import lax
from jax.experimental import pallas as pl
from jax.experimental.pallas import tpu as pltpu
from jax.experimental.pallas import tpu_sc as plsc

N_NODES = 10000
N_EDGES = 320000
D = 128

_EBLK = 2000
_NBLK = 2000
_GW = 128
_SW = 128
_ZR = 632


def _ln(x, g, b):
    mu = jnp.mean(x, axis=-1, keepdims=True)
    var = jnp.mean((x - mu) ** 2, axis=-1, keepdims=True)
    return (x - mu) * lax.rsqrt(var + 1e-5) * g + b


def _dot(x, w):
    return jnp.dot(x.astype(jnp.bfloat16), w.astype(jnp.bfloat16),
                   preferred_element_type=jnp.float32)


def _mlp(x, ws, bs, g=None, b=None):
    h = x
    n = len(ws)
    for i in range(n):
        h = _dot(h, ws[i]) + bs[i]
        if i < n - 1:
            h = jnp.maximum(h, 0.0)
    if g is not None:
        h = _ln(h, g, b)
    return h


def _full_spec(a):
    return pl.BlockSpec(a.shape, lambda i: tuple(0 for _ in a.shape))


def _row_spec(blk, width):
    return pl.BlockSpec((blk, width), lambda i: (i, 0))


def _enc_edge_tc(edge_features, p, rows, blk_off):
    ws = [jnp.asarray(w) for w in p["W"]]
    bs = [b.reshape(1, -1) for b in p["b"]]
    g = p["ln_g"].reshape(1, -1)
    bl = p["ln_b"].reshape(1, -1)

    def body(x_ref, w1, b1, w2, b2, w3, b3, gr, blr, o_ref):
        o_ref[...] = _mlp(x_ref[...], [w1[...], w2[...], w3[...]],
                          [b1[...], b2[...], b3[...]], gr[...], blr[...])

    consts = [ws[0], bs[0], ws[1], bs[1], ws[2], bs[2], g, bl]
    x_spec = pl.BlockSpec((_EBLK, edge_features.shape[1]),
                          lambda i: (i + blk_off, 0))
    return pl.pallas_call(
        body,
        grid=(rows // _EBLK,),
        in_specs=[x_spec] + [_full_spec(c) for c in consts],
        out_specs=_row_spec(_EBLK, D),
        out_shape=jax.ShapeDtypeStruct((rows, D), jnp.float32),
    )(edge_features, *consts)


def _enc_node_tc(node_features, p, w1a, w1b):
    ws = [jnp.asarray(w) for w in p["W"]]
    bs = [b.reshape(1, -1) for b in p["b"]]
    g = p["ln_g"].reshape(1, -1)
    bl = p["ln_b"].reshape(1, -1)

    def body(x_ref, w1, b1, w2, b2, w3, b3, gr, blr, wa, wb,
             lat_ref, ps_ref, pr_ref):
        lat = _mlp(x_ref[...], [w1[...], w2[...], w3[...]],
                   [b1[...], b2[...], b3[...]], gr[...], blr[...])
        lat_ref[...] = lat
        ps_ref[...] = _dot(lat, wa[...])
        pr_ref[...] = _dot(lat, wb[...])

    consts = [ws[0], bs[0], ws[1], bs[1], ws[2], bs[2], g, bl, w1a, w1b]
    return pl.pallas_call(
        body,
        grid=(N_NODES // _NBLK,),
        in_specs=[_row_spec(_NBLK, D)] + [_full_spec(c) for c in consts],
        out_specs=[_row_spec(_NBLK, D)] * 3,
        out_shape=[jax.ShapeDtypeStruct((N_NODES, D), jnp.float32)] * 3,
    )(node_features, *consts)


def _edge_update_tc(gs, gr, edge_lat, p, want_resid):
    w1c = jnp.asarray(p["W"][0][2 * D:])
    bs = [b.reshape(1, -1) for b in p["b"]]
    ws = [jnp.asarray(w) for w in p["W"]]
    g = p["ln_g"].reshape(1, -1)
    bl = p["ln_b"].reshape(1, -1)

    def body(gs_ref, gr_ref, el_ref, w1, b1, w2, b2, w3, b3, grf, blr, *outs):
        el = el_ref[...]
        h = (gs_ref[...] + gr_ref[...] + _dot(el, w1[...]) + b1[...])
        h = jnp.maximum(h, 0.0)
        h = jnp.maximum(_dot(h, w2[...]) + b2[...], 0.0)
        ne = _ln(_dot(h, w3[...]) + b3[...], grf[...], blr[...])
        outs[0][...] = ne
        if len(outs) > 1:
            outs[1][...] = el + ne

    rows = gs.shape[0]
    consts = [w1c, bs[0], ws[1], bs[1], ws[2], bs[2], g, bl]
    n_out = 2 if want_resid else 1
    return pl.pallas_call(
        body,
        grid=(rows // _EBLK,),
        in_specs=[_row_spec(_EBLK, D)] * 3 + [_full_spec(c) for c in consts],
        out_specs=[_row_spec(_EBLK, D)] * n_out,
        out_shape=[jax.ShapeDtypeStruct((rows, D), jnp.float32)] * n_out,
    )(gs, gr, edge_lat, *consts)


def _node_update_tc(node_lat, partials, p, next_proj, dec):
    wa = jnp.asarray(p["W"][0][:D])
    wb = jnp.asarray(p["W"][0][D:])
    bs = [b.reshape(1, -1) for b in p["b"]]
    ws = [jnp.asarray(w) for w in p["W"]]
    g = p["ln_g"].reshape(1, -1)
    bl = p["ln_b"].reshape(1, -1)

    consts = [wa, bs[0], wb, ws[1], bs[1], ws[2], bs[2], g, bl]
    if next_proj is not None:
        consts += list(next_proj)
        out_specs = [_row_spec(_NBLK, D)] * 3
        out_shape = [jax.ShapeDtypeStruct((N_NODES, D), jnp.float32)] * 3
    else:
        dws = [jnp.asarray(w) for w in dec["W"]]
        dbs = [b.reshape(1, -1) for b in dec["b"]]
        consts += [dws[0], dbs[0], dws[1], dbs[1], dws[2], dbs[2]]
        out_specs = [pl.BlockSpec((_NBLK, dws[2].shape[1]), lambda i: (i, 0))]
        out_shape = [jax.ShapeDtypeStruct((N_NODES, dws[2].shape[1]), jnp.float32)]

    n_parts = len(partials)

    def body(nl_ref, *args):
        pps = args[:n_parts]
        wa_r, b1, wb_r, w2, b2, w3, b3, grf, blr = args[n_parts:n_parts + 9]
        rest = args[n_parts + 9:]
        nl = nl_ref[...]
        agg = pps[0][0] + pps[0][1]
        for pp in pps[1:]:
            agg = agg + pp[0] + pp[1]
        h = _dot(nl, wa_r[...]) + _dot(agg, wb_r[...]) + b1[...]
        h = jnp.maximum(h, 0.0)
        h = jnp.maximum(_dot(h, w2[...]) + b2[...], 0.0)
        nn = _ln(_dot(h, w3[...]) + b3[...], grf[...], blr[...])
        node_next = nl + nn
        if next_proj is not None:
            wa2, wb2 = rest[0], rest[1]
            outs = rest[2:]
            outs[0][...] = node_next
            outs[1][...] = _dot(node_next, wa2[...])
            outs[2][...] = _dot(node_next, wb2[...])
        else:
            dw1, db1, dw2, db2, dw3, db3 = rest[:6]
            outs = rest[6:]
            h = jnp.maximum(_dot(node_next, dw1[...]) + db1[...], 0.0)
            h = jnp.maximum(_dot(h, dw2[...]) + db2[...], 0.0)
            outs[0][...] = _dot(h, dw3[...]) + db3[...]

    return pl.pallas_call(
        body,
        grid=(N_NODES // _NBLK,),
        in_specs=[_row_spec(_NBLK, D)]
                 + [pl.BlockSpec((2, _NBLK, D), lambda i: (0, i, 0))] * n_parts
                 + [_full_spec(c) for c in consts],
        out_specs=out_specs,
        out_shape=out_shape,
    )(node_lat, *partials, *consts)


def _sc_gather2(ps, pr, senders2d, receivers2d, ec, win_off):
    mesh = plsc.VectorSubcoreMesh(core_axis_name="core", subcore_axis_name="subcore")

    @functools.partial(
        pl.kernel,
        out_type=(jax.ShapeDtypeStruct((ec, D), jnp.float32),
                  jax.ShapeDtypeStruct((ec, D), jnp.float32)),
        mesh=mesh,
        scratch_types=[pltpu.VMEM_SHARED((N_NODES, D), jnp.float32)],
    )
    def k(ps_hbm, pr_hbm, si_hbm, ri_hbm, gs_hbm, gr_hbm, tab_sh):
        zr = 640
        cid = lax.axis_index("core")
        sid = lax.axis_index("subcore")
        base_n = jnp.minimum(sid * zr, N_NODES - zr)

        @pl.when(cid == 0)
        def _():
            pltpu.sync_copy(ps_hbm.at[pl.ds(base_n, zr)],
                            tab_sh.at[pl.ds(base_n, zr)])

        @pl.when(cid == 1)
        def _():
            pltpu.sync_copy(pr_hbm.at[pl.ds(base_n, zr)],
                            tab_sh.at[pl.ds(base_n, zr)])

        plsc.subcore_barrier()

        def body(i_v, o_v):
            pltpu.sync_copy(tab_sh.at[i_v.at[0]], o_v)

        def pipe(idx_hbm, out_hbm):
            pltpu.emit_pipeline(
                body,
                grid=(ec // _GW,),
                in_specs=[pl.BlockSpec((1, _GW), lambda i: (0, i + win_off))],
                out_specs=[pl.BlockSpec((_GW, D), lambda i: (i, 0))],
                core_axis_name="subcore",
                dimension_semantics=(pltpu.PARALLEL,),
            )(idx_hbm, out_hbm)

        @pl.when(cid == 0)
        def _():
            pipe(si_hbm, gs_hbm)

        @pl.when(cid == 1)
        def _():
            pipe(ri_hbm, gr_hbm)

    return k(ps, pr, senders2d, receivers2d)


def _sc_scatter_add(updates, receivers2d, zeros, win_off):
    ec = updates.shape[0]
    mesh = plsc.VectorSubcoreMesh(core_axis_name="core", subcore_axis_name="subcore")

    @functools.partial(
        pl.kernel,
        out_type=jax.ShapeDtypeStruct((2, N_NODES, D), jnp.float32),
        mesh=mesh,
        scratch_types=[
            pltpu.VMEM_SHARED((N_NODES, D), jnp.float32),
        ],
    )
    def k(upd_hbm, ri_hbm, z_hbm, out_hbm, acc_sh):
        cid = lax.axis_index("core")
        sid = lax.axis_index("subcore")
        base_n = jnp.minimum(sid * _ZR, N_NODES - _ZR)
        pltpu.sync_copy(z_hbm, acc_sh.at[pl.ds(base_n, _ZR)])
        plsc.subcore_barrier()

        def body(ri_v, upd_v):
            pltpu.sync_copy(upd_v, acc_sh.at[ri_v.at[0]], add=True)

        pltpu.emit_pipeline(
            body,
            grid=(ec // _SW,),
            in_specs=[pl.BlockSpec((1, _SW), lambda i: (0, i + win_off)),
                      pl.BlockSpec((_SW, D), lambda i: (i, 0))],
            out_specs=[],
            core_axis_name=("core", "subcore"),
            dimension_semantics=(pltpu.PARALLEL,),
        )(ri_hbm, upd_hbm)

        plsc.subcore_barrier()
        pltpu.sync_copy(acc_sh.at[pl.ds(base_n, _ZR)],
                        out_hbm.at[cid, pl.ds(base_n, _ZR)])

    return k(updates, receivers2d, zeros)


_CHUNKS = 4


def kernel(node_features, edge_features, senders, receivers, params):
    ec = N_EDGES // _CHUNKS
    s2d = senders.astype(jnp.int32).reshape(1, N_EDGES)
    r2d = receivers.astype(jnp.int32).reshape(1, N_EDGES)
    zeros = jnp.zeros((_ZR, D), jnp.float32)

    blocks = params["blocks"]
    w1a0 = jnp.asarray(blocks[0]["edge"]["W"][0][:D])
    w1b0 = jnp.asarray(blocks[0]["edge"]["W"][0][D:2 * D])

    edge_lat = [_enc_edge_tc(edge_features, params["enc_edge"], ec,
                             c * (ec // _EBLK)) for c in range(_CHUNKS)]
    node_lat, ps, pr = _enc_node_tc(node_features, params["enc_node"], w1a0, w1b0)

    n_steps = len(blocks)
    out = None
    for s in range(n_steps):
        blk = blocks[s]
        last = s == n_steps - 1
        partials = []
        for c in range(_CHUNKS):
            gs, gr = _sc_gather2(ps, pr, s2d, r2d, ec, c * (ec // _GW))
            if last:
                (new_edge,) = _edge_update_tc(gs, gr, edge_lat[c], blk["edge"], False)
            else:
                new_edge, edge_lat[c] = _edge_update_tc(gs, gr, edge_lat[c],
                                                        blk["edge"], True)
            partials.append(_sc_scatter_add(new_edge, r2d, zeros,
                                            c * (ec // _SW)))
        if last:
            (out,) = _node_update_tc(node_lat, partials, blk["node"], None,
                                     params["dec"])
        else:
            nxt = blocks[s + 1]["edge"]["W"][0]
            node_lat, ps, pr = _node_update_tc(
                node_lat, partials, blk["node"],
                (jnp.asarray(nxt[:D]), jnp.asarray(nxt[D:2 * D])), None)
    return out


# speedup vs baseline: 1.0008x; 1.0008x over previous
import functools

import jax
import jax.numpy as jnp
from jax import lax
from jax.experimental import pallas as pl
from jax.experimental.pallas import tpu as pltpu
from jax.experimental.pallas import tpu_sc as plsc

N_NODES = 10000
N_EDGES = 320000
D = 128

_EBLK = 2000
_NBLK = 2000
_GW = 128
_SW = 128
_ZR = 632


def _ln(x, g, b):
    mu = jnp.mean(x, axis=-1, keepdims=True)
    var = jnp.mean((x - mu) ** 2, axis=-1, keepdims=True)
    return (x - mu) * lax.rsqrt(var + 1e-5) * g + b


def _dot(x, w):
    return jnp.dot(x.astype(jnp.bfloat16), w.astype(jnp.bfloat16),
                   preferred_element_type=jnp.float32)


def _mlp(x, ws, bs, g=None, b=None):
    h = x
    n = len(ws)
    for i in range(n):
        h = _dot(h, ws[i]) + bs[i]
        if i < n - 1:
            h = jnp.maximum(h, 0.0)
    if g is not None:
        h = _ln(h, g, b)
    return h


def _full_spec(a):
    return pl.BlockSpec(a.shape, lambda i: tuple(0 for _ in a.shape))


def _row_spec(blk, width):
    return pl.BlockSpec((blk, width), lambda i: (i, 0))


def _enc_edge_tc(edge_features, p, rows, blk_off):
    ws = [jnp.asarray(w) for w in p["W"]]
    bs = [b.reshape(1, -1) for b in p["b"]]
    g = p["ln_g"].reshape(1, -1)
    bl = p["ln_b"].reshape(1, -1)

    def body(x_ref, w1, b1, w2, b2, w3, b3, gr, blr, o_ref):
        o_ref[...] = _mlp(x_ref[...], [w1[...], w2[...], w3[...]],
                          [b1[...], b2[...], b3[...]], gr[...], blr[...])

    consts = [ws[0], bs[0], ws[1], bs[1], ws[2], bs[2], g, bl]
    x_spec = pl.BlockSpec((_EBLK, edge_features.shape[1]),
                          lambda i: (i + blk_off, 0))
    return pl.pallas_call(
        body,
        grid=(rows // _EBLK,),
        in_specs=[x_spec] + [_full_spec(c) for c in consts],
        out_specs=_row_spec(_EBLK, D),
        out_shape=jax.ShapeDtypeStruct((rows, D), jnp.float32),
    )(edge_features, *consts)


def _enc_node_tc(node_features, p, w1a, w1b):
    ws = [jnp.asarray(w) for w in p["W"]]
    bs = [b.reshape(1, -1) for b in p["b"]]
    g = p["ln_g"].reshape(1, -1)
    bl = p["ln_b"].reshape(1, -1)

    def body(x_ref, w1, b1, w2, b2, w3, b3, gr, blr, wa, wb,
             lat_ref, ps_ref, pr_ref):
        lat = _mlp(x_ref[...], [w1[...], w2[...], w3[...]],
                   [b1[...], b2[...], b3[...]], gr[...], blr[...])
        lat_ref[...] = lat
        ps_ref[...] = _dot(lat, wa[...])
        pr_ref[...] = _dot(lat, wb[...])

    consts = [ws[0], bs[0], ws[1], bs[1], ws[2], bs[2], g, bl, w1a, w1b]
    return pl.pallas_call(
        body,
        grid=(N_NODES // _NBLK,),
        in_specs=[_row_spec(_NBLK, D)] + [_full_spec(c) for c in consts],
        out_specs=[_row_spec(_NBLK, D)] * 3,
        out_shape=[jax.ShapeDtypeStruct((N_NODES, D), jnp.float32)] * 3,
    )(node_features, *consts)


def _edge_update_tc(gs, gr, edge_lat, p, want_resid):
    w1c = jnp.asarray(p["W"][0][2 * D:])
    bs = [b.reshape(1, -1) for b in p["b"]]
    ws = [jnp.asarray(w) for w in p["W"]]
    g = p["ln_g"].reshape(1, -1)
    bl = p["ln_b"].reshape(1, -1)

    def body(gs_ref, gr_ref, el_ref, w1, b1, w2, b2, w3, b3, grf, blr, *outs):
        el = el_ref[...]
        h = (gs_ref[...] + gr_ref[...] + _dot(el, w1[...]) + b1[...])
        h = jnp.maximum(h, 0.0)
        h = jnp.maximum(_dot(h, w2[...]) + b2[...], 0.0)
        ne = _ln(_dot(h, w3[...]) + b3[...], grf[...], blr[...])
        outs[0][...] = ne
        if len(outs) > 1:
            outs[1][...] = el + ne

    rows = gs.shape[0]
    consts = [w1c, bs[0], ws[1], bs[1], ws[2], bs[2], g, bl]
    n_out = 2 if want_resid else 1
    return pl.pallas_call(
        body,
        grid=(rows // _EBLK,),
        in_specs=[_row_spec(_EBLK, D)] * 3 + [_full_spec(c) for c in consts],
        out_specs=[_row_spec(_EBLK, D)] * n_out,
        out_shape=[jax.ShapeDtypeStruct((rows, D), jnp.float32)] * n_out,
    )(gs, gr, edge_lat, *consts)


def _node_update_tc(node_lat, partials, p, next_proj, dec):
    wa = jnp.asarray(p["W"][0][:D])
    wb = jnp.asarray(p["W"][0][D:])
    bs = [b.reshape(1, -1) for b in p["b"]]
    ws = [jnp.asarray(w) for w in p["W"]]
    g = p["ln_g"].reshape(1, -1)
    bl = p["ln_b"].reshape(1, -1)

    consts = [wa, bs[0], wb, ws[1], bs[1], ws[2], bs[2], g, bl]
    if next_proj is not None:
        consts += list(next_proj)
        out_specs = [_row_spec(_NBLK, D)] * 3
        out_shape = [jax.ShapeDtypeStruct((N_NODES, D), jnp.float32)] * 3
    else:
        dws = [jnp.asarray(w) for w in dec["W"]]
        dbs = [b.reshape(1, -1) for b in dec["b"]]
        consts += [dws[0], dbs[0], dws[1], dbs[1], dws[2], dbs[2]]
        out_specs = [pl.BlockSpec((_NBLK, dws[2].shape[1]), lambda i: (i, 0))]
        out_shape = [jax.ShapeDtypeStruct((N_NODES, dws[2].shape[1]), jnp.float32)]

    n_parts = len(partials)

    def body(nl_ref, *args):
        pps = args[:n_parts]
        wa_r, b1, wb_r, w2, b2, w3, b3, grf, blr = args[n_parts:n_parts + 9]
        rest = args[n_parts + 9:]
        nl = nl_ref[...]
        agg = pps[0][0] + pps[0][1]
        for pp in pps[1:]:
            agg = agg + pp[0] + pp[1]
        h = _dot(nl, wa_r[...]) + _dot(agg, wb_r[...]) + b1[...]
        h = jnp.maximum(h, 0.0)
        h = jnp.maximum(_dot(h, w2[...]) + b2[...], 0.0)
        nn = _ln(_dot(h, w3[...]) + b3[...], grf[...], blr[...])
        node_next = nl + nn
        if next_proj is not None:
            wa2, wb2 = rest[0], rest[1]
            outs = rest[2:]
            outs[0][...] = node_next
            outs[1][...] = _dot(node_next, wa2[...])
            outs[2][...] = _dot(node_next, wb2[...])
        else:
            dw1, db1, dw2, db2, dw3, db3 = rest[:6]
            outs = rest[6:]
            h = jnp.maximum(_dot(node_next, dw1[...]) + db1[...], 0.0)
            h = jnp.maximum(_dot(h, dw2[...]) + db2[...], 0.0)
            outs[0][...] = _dot(h, dw3[...]) + db3[...]

    return pl.pallas_call(
        body,
        grid=(N_NODES // _NBLK,),
        in_specs=[_row_spec(_NBLK, D)]
                 + [pl.BlockSpec((2, _NBLK, D), lambda i: (0, i, 0))] * n_parts
                 + [_full_spec(c) for c in consts],
        out_specs=out_specs,
        out_shape=out_shape,
    )(node_lat, *partials, *consts)


def _sc_gather2(ps, pr, senders2d, receivers2d, ec, win_off):
    mesh = plsc.VectorSubcoreMesh(core_axis_name="core", subcore_axis_name="subcore")

    @functools.partial(
        pl.kernel,
        out_type=(jax.ShapeDtypeStruct((ec, D), jnp.float32),
                  jax.ShapeDtypeStruct((ec, D), jnp.float32)),
        mesh=mesh,
        scratch_types=[pltpu.VMEM_SHARED((N_NODES, D), jnp.float32)],
    )
    def k(ps_hbm, pr_hbm, si_hbm, ri_hbm, gs_hbm, gr_hbm, tab_sh):
        zr = 640
        cid = lax.axis_index("core")
        sid = lax.axis_index("subcore")
        base_n = jnp.minimum(sid * zr, N_NODES - zr)

        @pl.when(cid == 0)
        def _():
            pltpu.sync_copy(ps_hbm.at[pl.ds(base_n, zr)],
                            tab_sh.at[pl.ds(base_n, zr)])

        @pl.when(cid == 1)
        def _():
            pltpu.sync_copy(pr_hbm.at[pl.ds(base_n, zr)],
                            tab_sh.at[pl.ds(base_n, zr)])

        plsc.subcore_barrier()

        def body(i_v, o_v):
            pltpu.sync_copy(tab_sh.at[i_v.at[0]], o_v)

        def pipe(idx_hbm, out_hbm):
            pltpu.emit_pipeline(
                body,
                grid=(ec // _GW,),
                in_specs=[pl.BlockSpec((1, _GW), lambda i: (0, i + win_off))],
                out_specs=[pl.BlockSpec((_GW, D), lambda i: (i, 0))],
                core_axis_name="subcore",
                dimension_semantics=(pltpu.PARALLEL,),
            )(idx_hbm, out_hbm)

        @pl.when(cid == 0)
        def _():
            pipe(si_hbm, gs_hbm)

        @pl.when(cid == 1)
        def _():
            pipe(ri_hbm, gr_hbm)

    return k(ps, pr, senders2d, receivers2d)


def _sc_scatter_add(updates, receivers2d, zeros, win_off):
    ec = updates.shape[0]
    mesh = plsc.VectorSubcoreMesh(core_axis_name="core", subcore_axis_name="subcore")

    @functools.partial(
        pl.kernel,
        out_type=jax.ShapeDtypeStruct((2, N_NODES, D), jnp.float32),
        mesh=mesh,
        scratch_types=[
            pltpu.VMEM_SHARED((N_NODES, D), jnp.float32),
        ],
    )
    def k(upd_hbm, ri_hbm, z_hbm, out_hbm, acc_sh):
        cid = lax.axis_index("core")
        sid = lax.axis_index("subcore")
        base_n = jnp.minimum(sid * _ZR, N_NODES - _ZR)
        pltpu.sync_copy(z_hbm, acc_sh.at[pl.ds(base_n, _ZR)])
        plsc.subcore_barrier()

        def body(ri_v, upd_v):
            pltpu.sync_copy(upd_v, acc_sh.at[ri_v.at[0]], add=True)

        pltpu.emit_pipeline(
            body,
            grid=(ec // _SW,),
            in_specs=[pl.BlockSpec((1, _SW), lambda i: (0, i + win_off)),
                      pl.BlockSpec((_SW, D), lambda i: (i, 0))],
            out_specs=[],
            core_axis_name=("core", "subcore"),
            dimension_semantics=(pltpu.PARALLEL,),
        )(ri_hbm, upd_hbm)

        plsc.subcore_barrier()
        pltpu.sync_copy(acc_sh.at[pl.ds(base_n, _ZR)],
                        out_hbm.at[cid, pl.ds(base_n, _ZR)])

    return k(updates, receivers2d, zeros)


_CHUNKS = 2


def kernel(node_features, edge_features, senders, receivers, params):
    ec = N_EDGES // _CHUNKS
    s2d = senders.astype(jnp.int32).reshape(1, N_EDGES)
    r2d = receivers.astype(jnp.int32).reshape(1, N_EDGES)
    zeros = jnp.zeros((_ZR, D), jnp.float32)

    blocks = params["blocks"]
    w1a0 = jnp.asarray(blocks[0]["edge"]["W"][0][:D])
    w1b0 = jnp.asarray(blocks[0]["edge"]["W"][0][D:2 * D])

    edge_lat = [_enc_edge_tc(edge_features, params["enc_edge"], ec,
                             c * (ec // _EBLK)) for c in range(_CHUNKS)]
    node_lat, ps, pr = _enc_node_tc(node_features, params["enc_node"], w1a0, w1b0)

    n_steps = len(blocks)
    out = None
    for s in range(n_steps):
        blk = blocks[s]
        last = s == n_steps - 1
        partials = []
        for c in range(_CHUNKS):
            gs, gr = _sc_gather2(ps, pr, s2d, r2d, ec, c * (ec // _GW))
            if last:
                (new_edge,) = _edge_update_tc(gs, gr, edge_lat[c], blk["edge"], False)
            else:
                new_edge, edge_lat[c] = _edge_update_tc(gs, gr, edge_lat[c],
                                                        blk["edge"], True)
            partials.append(_sc_scatter_add(new_edge, r2d, zeros,
                                            c * (ec // _SW)))
        if last:
            (out,) = _node_update_tc(node_lat, partials, blk["node"], None,
                                     params["dec"])
        else:
            nxt = blocks[s + 1]["edge"]["W"][0]
            node_lat, ps, pr = _node_update_tc(
                node_lat, partials, blk["node"],
                (jnp.asarray(nxt[:D]), jnp.asarray(nxt[D:2 * D])), None)
    return out
